# baseline (speedup 1.0000x reference)
import math

import jax
import jax.numpy as jnp
from jax import lax
from jax.experimental import pallas as pl
from jax.experimental.pallas import tpu as pltpu
from jax.experimental.pallas import tpu_sc as plsc

LAT = 721
LON = 1440
LEVEL = 4
INV_RES = 4.0
N_ROWS = LAT * LON
LAT_PAD = 728
LON_PAD = 1536
NCB = LON_PAD // 128
CB_SZ = LAT_PAD * 128
PLANE = NCB * CB_SZ

NC = 2
NS = 16
NW = NC * NS
LANES = 16

BLK = 1600
BLK_PAD = 1664
STAGE_ROWS = 16
NCB_STAGE = STAGE_ROWS
STAGE_SZ = NCB * STAGE_ROWS * 128
MAGIC = 12582912.0


def _interp_matrix(src, dst):
    return jax.image.resize(jnp.eye(src, dtype=jnp.float32), (dst, src),
                            method="bilinear")


def _table_body(g0, g1, g2, g3, r0, r1, c1t, r2, c2t, r3, c3t, npb, spb,
                out_ref):
    rows = lax.broadcasted_iota(jnp.int32, (LAT_PAD, LON_PAD), 0)
    g0p = jnp.concatenate(
        [g0[...], jnp.zeros((LAT, LON_PAD - LON), jnp.float32)], axis=1)
    a = [g0p,
         jnp.dot(g1[...], c1t[...], preferred_element_type=jnp.float32),
         jnp.dot(g2[...], c2t[...], preferred_element_type=jnp.float32),
         jnp.dot(g3[...], c3t[...], preferred_element_type=jnp.float32)]
    rs = [r0, r1, r2, r3]
    for l in range(LEVEL):
        u = jnp.dot(rs[l][...], a[l], preferred_element_type=jnp.float32)
        u = jnp.where(rows == 0, spb[l:l + 1, :], u)
        u = jnp.where(rows == LAT - 1, npb[l:l + 1, :], u)
        for cb in range(NCB):
            out_ref[pl.ds((l * NCB + cb) * LAT_PAD, LAT_PAD), :] = (
                u[:, 128 * cb:128 * (cb + 1)])


def _build_table(g0, g1, g2, g3, north, south):
    shapes = [(int(math.ceil(LAT / 2 ** i)), int(math.ceil(LON / 2 ** i)))
              for i in range(LEVEL)]
    pad = jnp.zeros((LAT_PAD - LAT, LAT), jnp.float32)
    mats = [jnp.concatenate([jnp.eye(LAT, dtype=jnp.float32), pad], 0)]
    for i in (1, 2, 3):
        h, w = shapes[i]
        r = _interp_matrix(h, LAT)
        mats.append(jnp.concatenate(
            [r, jnp.zeros((LAT_PAD - LAT, h), jnp.float32)], 0))
        ct = _interp_matrix(w, LON).T
        mats.append(jnp.concatenate(
            [ct, jnp.zeros((w, LON_PAD - LON), jnp.float32)], 1))
    npb = jnp.broadcast_to(north[:, None], (LEVEL, LON_PAD))
    spb = jnp.broadcast_to(south[:, None], (LEVEL, LON_PAD))
    return pl.pallas_call(
        _table_body,
        out_shape=jax.ShapeDtypeStruct((LEVEL * NCB * LAT_PAD, 128),
                                       jnp.float32),
    )(g0, g1, g2, g3, *mats, npb, spb)


def _point_rows(latv):
    v = (90.0 - latv) * INV_RES
    r = (v + MAGIC) - MAGIC
    r = jnp.minimum(jnp.maximum(r, 0.0), float(LAT - 1))
    return r.astype(jnp.int32)


def _point_cols(lonv):
    v = lonv * INV_RES
    r = (v + MAGIC) - MAGIC
    r = jnp.minimum(jnp.maximum(r, 0.0), float(LON - 1))
    return r.astype(jnp.int32)


def _sc_gather_body(x_hbm, tab_hbm, out_hbm,
                    xv, staged, idxv4, rows4, rowsv, rmin_v, rmax_v, sem):
    wid = lax.axis_index("s") * NC + lax.axis_index("c")
    n_blocks = x_hbm.shape[0] // (2 * BLK)
    blocks_per_tile = (n_blocks + NW - 1) // NW
    iota = lax.iota(jnp.int32, LANES)
    zeros = jnp.zeros((LANES,), jnp.int32)
    ones = jnp.full((LANES,), 1, jnp.int32)

    rmin_v[...] = jnp.full((LANES,), LAT - 1, jnp.int32)
    rmax_v[...] = jnp.zeros((LANES,), jnp.int32)

    def p1(i, carry):
        blk = i * NW + wid

        @pl.when(blk < n_blocks)
        def _():
            pltpu.sync_copy(x_hbm.at[pl.ds(blk * (2 * BLK), 2 * BLK)], xv)
            for j in range(BLK // LANES):
                latv = plsc.load_gather(xv, [2 * LANES * j + 2 * iota])
                ri = _point_rows(latv)
                rmin_v[...] = jnp.minimum(rmin_v[...], ri)
                rmax_v[...] = jnp.maximum(rmax_v[...], ri)

        return carry

    lax.fori_loop(0, blocks_per_tile, p1, 0)
    lo = jnp.minimum(jnp.min(rmin_v[...]), LAT - STAGE_ROWS)
    hi = jnp.max(rmax_v[...])
    fast = (hi - lo) < STAGE_ROWS

    @pl.when(fast)
    def _():
        copies = [
            pltpu.async_copy(
                tab_hbm.at[pl.ds((l * NCB + cb) * CB_SZ + lo * 128,
                                 STAGE_ROWS * 128)],
                staged.at[pl.ds((l * NCB + cb) * STAGE_ROWS * 128,
                                STAGE_ROWS * 128)], sem)
            for l in range(LEVEL) for cb in range(NCB)
        ]
        for c in copies:
            c.wait()

        def block_fast(i, carry):
            blk = i * NW + wid

            @pl.when(blk < n_blocks)
            def _():
                base = blk * BLK
                pltpu.sync_copy(x_hbm.at[pl.ds(2 * base, 2 * BLK)], xv)
                for j in range(BLK // LANES):
                    pidx = LANES * j + iota
                    pos = 2 * LANES * j + 2 * iota
                    ri = _point_rows(plsc.load_gather(xv, [pos]))
                    ci = _point_cols(plsc.load_gather(xv, [pos + 1]))
                    cb = lax.shift_right_logical(ci, 7)
                    cr = jnp.bitwise_and(ci, 127)
                    loc = (cb * NCB_STAGE + (ri - lo)) * 128 + cr
                    for l in range(LEVEL):
                        v = plsc.load_gather(staged, [loc + l * STAGE_SZ])
                        plsc.store_scatter(rowsv, [LEVEL * pidx + l], v)
                pltpu.sync_copy(rowsv.at[pl.ds(0, LEVEL * BLK)],
                                out_hbm.at[pl.ds(LEVEL * base, LEVEL * BLK)])

            return carry

        lax.fori_loop(0, blocks_per_tile, block_fast, 0)

    @pl.when(jnp.logical_not(fast))
    def _():
        def pad(t, carry):
            for l in range(LEVEL):
                idxv4[pl.ds(BLK_PAD * l + BLK + LANES * t, LANES)] = zeros
            return carry

        lax.fori_loop(0, (BLK_PAD - BLK) // LANES, pad, 0)

        def block_slow(i, carry):
            blk = i * NW + wid

            @pl.when(blk < n_blocks)
            def _():
                base = blk * BLK
                pltpu.sync_copy(x_hbm.at[pl.ds(2 * base, 2 * BLK)], xv)

                def jbody(j, c):
                    pidx = LANES * j + iota
                    pos = 2 * LANES * j + 2 * iota
                    ri = _point_rows(plsc.load_gather(xv, [pos]))
                    ci = _point_cols(plsc.load_gather(xv, [pos + 1]))
                    cb = lax.shift_right_logical(ci, 7)
                    cr = jnp.bitwise_and(ci, 127)
                    fidx = (cb * LAT_PAD + ri) * 128 + cr
                    for l in range(LEVEL):
                        idxv4[pl.ds(BLK_PAD * l + LANES * j, LANES)] = (
                            fidx + l * PLANE)
                    return c

                lax.fori_loop(0, BLK // LANES, jbody, 0)
                for l in range(LEVEL):
                    copies = [
                        pltpu.async_copy(
                            tab_hbm.at[idxv4.at[pl.ds(BLK_PAD * l + 128 * k,
                                                      128)]],
                            rows4.at[pl.ds(BLK_PAD * l + 128 * k, 128)], sem)
                        for k in range(BLK_PAD // 128)
                    ]
                    for c in copies:
                        c.wait()

                def abody(j, c):
                    pidx = LANES * j + iota
                    for l in range(LEVEL):
                        v = plsc.load_gather(rows4, [BLK_PAD * l + pidx])
                        plsc.store_scatter(rowsv, [LEVEL * pidx + l], v)
                    return c

                lax.fori_loop(0, BLK // LANES, abody, 0)
                pltpu.sync_copy(rowsv.at[pl.ds(0, LEVEL * BLK)],
                                out_hbm.at[pl.ds(LEVEL * base, LEVEL * BLK)])

            return carry

        lax.fori_loop(0, blocks_per_tile, block_slow, 0)


def _sc_gather(x, tab1d, n_points):
    return pl.kernel(
        _sc_gather_body,
        out_type=jax.ShapeDtypeStruct((n_points * LEVEL,), jnp.float32),
        mesh=plsc.VectorSubcoreMesh(core_axis_name="c", subcore_axis_name="s"),
        compiler_params=pltpu.CompilerParams(needs_layout_passes=False),
        scratch_types=[
            pltpu.VMEM((2 * BLK,), jnp.float32),
            pltpu.VMEM((LEVEL * STAGE_SZ,), jnp.float32),
            pltpu.VMEM((BLK_PAD * LEVEL,), jnp.int32),
            pltpu.VMEM((BLK_PAD * LEVEL,), jnp.float32),
            pltpu.VMEM((BLK_PAD * LEVEL,), jnp.float32),
            pltpu.VMEM((LANES,), jnp.int32),
            pltpu.VMEM((LANES,), jnp.int32),
            pltpu.SemaphoreType.DMA,
        ],
    )(x, tab1d)


def kernel(x, grid0, grid1, grid2, grid3, north_pole_param, south_pole_param):
    n_points = x.shape[0]
    planar = _build_table(grid0[0, 0], grid1[0, 0], grid2[0, 0], grid3[0, 0],
                          north_pole_param, south_pole_param)
    flat = _sc_gather(x.reshape(-1), planar.reshape(-1), n_points)
    return flat.reshape(n_points, LEVEL)


# speedup vs baseline: 9.0695x; 9.0695x over previous
import math

import jax
import jax.numpy as jnp
from jax import lax
from jax.experimental import pallas as pl
from jax.experimental.pallas import tpu as pltpu
from jax.experimental.pallas import tpu_sc as plsc

LAT = 721
LON = 1440
LEVEL = 4
INV_RES = 4.0
N_ROWS = LAT * LON
LAT_PAD = 728
LON_PAD = 1536
NCB = LON_PAD // 128
CB_SZ = LAT_PAD * 128
PLANE = NCB * CB_SZ

NC = 2
NS = 16
NW = NC * NS
LANES = 16

BLK = 1600
BLK_PAD = 1664
STAGE_ROWS = 16
NCB_STAGE = STAGE_ROWS
STAGE_SZ = NCB * STAGE_ROWS * 128
MAGIC = 12582912.0


def _interp_matrix(src, dst):
    return jax.image.resize(jnp.eye(src, dtype=jnp.float32), (dst, src),
                            method="bilinear")


def _table_body(g0, g1, g2, g3, r0, r1, c1t, r2, c2t, r3, c3t, npb, spb,
                out_ref):
    rows = lax.broadcasted_iota(jnp.int32, (LAT_PAD, LON_PAD), 0)
    g0p = jnp.concatenate(
        [g0[...], jnp.zeros((LAT, LON_PAD - LON), jnp.float32)], axis=1)
    a = [g0p,
         jnp.dot(g1[...], c1t[...], preferred_element_type=jnp.float32),
         jnp.dot(g2[...], c2t[...], preferred_element_type=jnp.float32),
         jnp.dot(g3[...], c3t[...], preferred_element_type=jnp.float32)]
    rs = [r0, r1, r2, r3]
    for l in range(LEVEL):
        u = jnp.dot(rs[l][...], a[l], preferred_element_type=jnp.float32)
        u = jnp.where(rows == 0, spb[l:l + 1, :], u)
        u = jnp.where(rows == LAT - 1, npb[l:l + 1, :], u)
        for cb in range(NCB):
            out_ref[pl.ds((l * NCB + cb) * LAT_PAD, LAT_PAD), :] = (
                u[:, 128 * cb:128 * (cb + 1)])


def _build_table(g0, g1, g2, g3, north, south):
    shapes = [(int(math.ceil(LAT / 2 ** i)), int(math.ceil(LON / 2 ** i)))
              for i in range(LEVEL)]
    pad = jnp.zeros((LAT_PAD - LAT, LAT), jnp.float32)
    mats = [jnp.concatenate([jnp.eye(LAT, dtype=jnp.float32), pad], 0)]
    for i in (1, 2, 3):
        h, w = shapes[i]
        r = _interp_matrix(h, LAT)
        mats.append(jnp.concatenate(
            [r, jnp.zeros((LAT_PAD - LAT, h), jnp.float32)], 0))
        ct = _interp_matrix(w, LON).T
        mats.append(jnp.concatenate(
            [ct, jnp.zeros((w, LON_PAD - LON), jnp.float32)], 1))
    npb = jnp.broadcast_to(north[:, None], (LEVEL, LON_PAD))
    spb = jnp.broadcast_to(south[:, None], (LEVEL, LON_PAD))
    return pl.pallas_call(
        _table_body,
        out_shape=jax.ShapeDtypeStruct((LEVEL * NCB * LAT_PAD, 128),
                                       jnp.float32),
    )(g0, g1, g2, g3, *mats, npb, spb)


def _point_rows(latv):
    v = (90.0 - latv) * INV_RES
    r = (v + MAGIC) - MAGIC
    r = jnp.minimum(jnp.maximum(r, 0.0), float(LAT - 1))
    return r.astype(jnp.int32)


def _point_cols(lonv):
    v = lonv * INV_RES
    r = (v + MAGIC) - MAGIC
    r = jnp.minimum(jnp.maximum(r, 0.0), float(LON - 1))
    return r.astype(jnp.int32)


def _sc_gather_body(x_hbm, tab_hbm, out_hbm,
                    xv, staged, idxv4, rows4, rowsv, rmin_v, rmax_v, sem):
    wid = lax.axis_index("s") * NC + lax.axis_index("c")
    n_points = x_hbm.shape[0] // 2
    n_blocks = n_points // BLK
    blocks_per_tile = (n_blocks + NW - 1) // NW
    iota = lax.iota(jnp.int32, LANES)
    zeros = jnp.zeros((LANES,), jnp.int32)

    rmin_v[...] = jnp.full((LANES,), LAT - 1, jnp.int32)
    rmax_v[...] = jnp.zeros((LANES,), jnp.int32)

    def p1(i, carry):
        blk = i * NW + wid

        @pl.when(blk < n_blocks)
        def _():
            pltpu.sync_copy(x_hbm.at[pl.ds(blk * BLK, BLK)],
                            xv.at[pl.ds(0, BLK)])
            for j in range(BLK // LANES):
                ri = _point_rows(xv[pl.ds(LANES * j, LANES)])
                rmin_v[...] = jnp.minimum(rmin_v[...], ri)
                rmax_v[...] = jnp.maximum(rmax_v[...], ri)

        return carry

    lax.fori_loop(0, blocks_per_tile, p1, 0)
    lo = jnp.minimum(jnp.min(rmin_v[...]), LAT - STAGE_ROWS)
    hi = jnp.max(rmax_v[...])
    fast = (hi - lo) < STAGE_ROWS

    @pl.when(fast)
    def _():
        copies = [
            pltpu.async_copy(
                tab_hbm.at[pl.ds((l * NCB + cb) * CB_SZ + lo * 128,
                                 STAGE_ROWS * 128)],
                staged.at[pl.ds((l * NCB + cb) * STAGE_ROWS * 128,
                                STAGE_ROWS * 128)], sem)
            for l in range(LEVEL) for cb in range(NCB)
        ]
        for c in copies:
            c.wait()

        def block_fast(i, carry):
            blk = i * NW + wid

            @pl.when(blk < n_blocks)
            def _():
                base = blk * BLK
                pltpu.sync_copy(x_hbm.at[pl.ds(base, BLK)],
                                xv.at[pl.ds(0, BLK)])
                pltpu.sync_copy(x_hbm.at[pl.ds(n_points + base, BLK)],
                                xv.at[pl.ds(BLK, BLK)])
                for j in range(BLK // LANES):
                    ri = _point_rows(xv[pl.ds(LANES * j, LANES)])
                    ci = _point_cols(xv[pl.ds(BLK + LANES * j, LANES)])
                    cb = lax.shift_right_logical(ci, 7)
                    cr = jnp.bitwise_and(ci, 127)
                    loc = (cb * STAGE_ROWS + (ri - lo)) * 128 + cr
                    for l in range(LEVEL):
                        v = plsc.load_gather(staged, [loc + l * STAGE_SZ])
                        rowsv[pl.ds(l * BLK + LANES * j, LANES)] = v
                for l in range(LEVEL):
                    pltpu.sync_copy(
                        rowsv.at[pl.ds(l * BLK, BLK)],
                        out_hbm.at[pl.ds(l * n_points + base, BLK)])

            return carry

        lax.fori_loop(0, blocks_per_tile, block_fast, 0)

    @pl.when(jnp.logical_not(fast))
    def _():
        def pad(t, carry):
            for l in range(LEVEL):
                idxv4[pl.ds(BLK_PAD * l + BLK + LANES * t, LANES)] = zeros
            return carry

        lax.fori_loop(0, (BLK_PAD - BLK) // LANES, pad, 0)

        def block_slow(i, carry):
            blk = i * NW + wid

            @pl.when(blk < n_blocks)
            def _():
                base = blk * BLK
                pltpu.sync_copy(x_hbm.at[pl.ds(base, BLK)],
                                xv.at[pl.ds(0, BLK)])
                pltpu.sync_copy(x_hbm.at[pl.ds(n_points + base, BLK)],
                                xv.at[pl.ds(BLK, BLK)])

                def jbody(j, c):
                    ri = _point_rows(xv[pl.ds(LANES * j, LANES)])
                    ci = _point_cols(xv[pl.ds(BLK + LANES * j, LANES)])
                    cb = lax.shift_right_logical(ci, 7)
                    cr = jnp.bitwise_and(ci, 127)
                    fidx = (cb * LAT_PAD + ri) * 128 + cr
                    for l in range(LEVEL):
                        idxv4[pl.ds(BLK_PAD * l + LANES * j, LANES)] = (
                            fidx + l * PLANE)
                    return c

                lax.fori_loop(0, BLK // LANES, jbody, 0)
                for l in range(LEVEL):
                    copies = [
                        pltpu.async_copy(
                            tab_hbm.at[idxv4.at[pl.ds(BLK_PAD * l + 128 * k,
                                                      128)]],
                            rows4.at[pl.ds(BLK_PAD * l + 128 * k, 128)], sem)
                        for k in range(BLK_PAD // 128)
                    ]
                    for c in copies:
                        c.wait()
                for l in range(LEVEL):
                    pltpu.sync_copy(
                        rows4.at[pl.ds(l * BLK_PAD, BLK)],
                        out_hbm.at[pl.ds(l * n_points + base, BLK)])

            return carry

        lax.fori_loop(0, blocks_per_tile, block_slow, 0)


def _sc_gather(x, tab1d, n_points):
    return pl.kernel(
        _sc_gather_body,
        out_type=jax.ShapeDtypeStruct((n_points * LEVEL,), jnp.float32),
        mesh=plsc.VectorSubcoreMesh(core_axis_name="c", subcore_axis_name="s"),
        compiler_params=pltpu.CompilerParams(needs_layout_passes=False),
        scratch_types=[
            pltpu.VMEM((2 * BLK,), jnp.float32),
            pltpu.VMEM((LEVEL * STAGE_SZ,), jnp.float32),
            pltpu.VMEM((BLK_PAD * LEVEL,), jnp.int32),
            pltpu.VMEM((BLK_PAD * LEVEL,), jnp.float32),
            pltpu.VMEM((BLK_PAD * LEVEL,), jnp.float32),
            pltpu.VMEM((LANES,), jnp.int32),
            pltpu.VMEM((LANES,), jnp.int32),
            pltpu.SemaphoreType.DMA,
        ],
    )(x, tab1d)


def kernel(x, grid0, grid1, grid2, grid3, north_pole_param, south_pole_param):
    n_points = x.shape[0]
    planar = _build_table(grid0[0, 0], grid1[0, 0], grid2[0, 0], grid3[0, 0],
                          north_pole_param, south_pole_param)
    flat = _sc_gather(x.T.reshape(-1), planar.reshape(-1), n_points)
    return flat.reshape(LEVEL, n_points).T


# speedup vs baseline: 10.0159x; 1.1043x over previous
import math

import numpy as np

import jax
import jax.numpy as jnp
from jax import lax
from jax.experimental import pallas as pl
from jax.experimental.pallas import tpu as pltpu
from jax.experimental.pallas import tpu_sc as plsc

LAT = 721
LON = 1440
LEVEL = 4
INV_RES = 4.0
N_ROWS = LAT * LON
LAT_PAD = 728
LON_PAD = 1536
NCB = LON_PAD // 128
CB_SZ = LAT_PAD * 128
PLANE = NCB * CB_SZ

NC = 2
NS = 16
NW = NC * NS
LANES = 16

BLK = 1600
BLK_PAD = 1664
STAGE_ROWS = 16
NCB_STAGE = STAGE_ROWS
STAGE_SZ = NCB * STAGE_ROWS * 128
MAGIC = 12582912.0


def _interp_matrix(src, dst):
    scale = dst / src
    sample = (np.arange(dst, dtype=np.float64) + 0.5) / scale - 0.5
    j = np.arange(src, dtype=np.float64)[None, :]
    w = np.maximum(0.0, 1.0 - np.abs(sample[:, None] - j))
    w = w / np.sum(w, axis=1, keepdims=True)
    return w.astype(np.float32)


def _table_body(g0, g1, g2, g3, r1, c1t, r2, c2t, r3, c3t, npb, spb,
                out_ref):
    rows = lax.broadcasted_iota(jnp.int32, (LAT_PAD, LON_PAD), 0)
    u0 = jnp.concatenate(
        [jnp.concatenate(
            [g0[...], jnp.zeros((LAT, LON_PAD - LON), jnp.float32)], axis=1),
         jnp.zeros((LAT_PAD - LAT, LON_PAD), jnp.float32)], axis=0)
    us = [u0,
          jnp.dot(r1[...],
                  jnp.dot(g1[...], c1t[...],
                          preferred_element_type=jnp.float32),
                  preferred_element_type=jnp.float32),
          jnp.dot(r2[...],
                  jnp.dot(g2[...], c2t[...],
                          preferred_element_type=jnp.float32),
                  preferred_element_type=jnp.float32),
          jnp.dot(r3[...],
                  jnp.dot(g3[...], c3t[...],
                          preferred_element_type=jnp.float32),
                  preferred_element_type=jnp.float32)]
    for l in range(LEVEL):
        u = jnp.where(rows == 0, spb[l:l + 1, :], us[l])
        u = jnp.where(rows == LAT - 1, npb[l:l + 1, :], u)
        for cb in range(NCB):
            out_ref[pl.ds((l * NCB + cb) * LAT_PAD, LAT_PAD), :] = (
                u[:, 128 * cb:128 * (cb + 1)])


def _build_table(g0, g1, g2, g3, north, south):
    shapes = [(int(math.ceil(LAT / 2 ** i)), int(math.ceil(LON / 2 ** i)))
              for i in range(LEVEL)]
    mats = []
    for i in (1, 2, 3):
        h, w = shapes[i]
        r = np.zeros((LAT_PAD, h), np.float32)
        r[:LAT] = _interp_matrix(h, LAT)
        mats.append(r)
        ct = np.zeros((w, LON_PAD), np.float32)
        ct[:, :LON] = _interp_matrix(w, LON).T
        mats.append(ct)
    npb = jnp.broadcast_to(north[:, None], (LEVEL, LON_PAD))
    spb = jnp.broadcast_to(south[:, None], (LEVEL, LON_PAD))
    return pl.pallas_call(
        _table_body,
        out_shape=jax.ShapeDtypeStruct((LEVEL * NCB * LAT_PAD, 128),
                                       jnp.float32),
    )(g0, g1, g2, g3, *mats, npb, spb)


def _point_rows(latv):
    v = (90.0 - latv) * INV_RES
    r = (v + MAGIC) - MAGIC
    r = jnp.minimum(jnp.maximum(r, 0.0), float(LAT - 1))
    return r.astype(jnp.int32)


def _point_cols(lonv):
    v = lonv * INV_RES
    r = (v + MAGIC) - MAGIC
    r = jnp.minimum(jnp.maximum(r, 0.0), float(LON - 1))
    return r.astype(jnp.int32)


def _sc_gather_body(x_hbm, tab_hbm, out_hbm,
                    xv, staged, idxv4, rows4, rowsv, rmin_v, rmax_v, sem):
    wid = lax.axis_index("s") * NC + lax.axis_index("c")
    n_points = x_hbm.shape[0] // 2
    n_blocks = n_points // BLK
    blocks_per_tile = (n_blocks + NW - 1) // NW
    iota = lax.iota(jnp.int32, LANES)
    zeros = jnp.zeros((LANES,), jnp.int32)

    rmin_v[...] = jnp.full((LANES,), LAT - 1, jnp.int32)
    rmax_v[...] = jnp.zeros((LANES,), jnp.int32)

    def p1(i, carry):
        blk = i * NW + wid

        @pl.when(blk < n_blocks)
        def _():
            pltpu.sync_copy(x_hbm.at[pl.ds(blk * BLK, BLK)],
                            xv.at[pl.ds(0, BLK)])
            for j in range(BLK // LANES):
                ri = _point_rows(xv[pl.ds(LANES * j, LANES)])
                rmin_v[...] = jnp.minimum(rmin_v[...], ri)
                rmax_v[...] = jnp.maximum(rmax_v[...], ri)

        return carry

    lax.fori_loop(0, blocks_per_tile, p1, 0)
    lo = jnp.minimum(jnp.min(rmin_v[...]), LAT - STAGE_ROWS)
    hi = jnp.max(rmax_v[...])
    fast = (hi - lo) < STAGE_ROWS

    @pl.when(fast)
    def _():
        copies = [
            pltpu.async_copy(
                tab_hbm.at[pl.ds((l * NCB + cb) * CB_SZ + lo * 128,
                                 STAGE_ROWS * 128)],
                staged.at[pl.ds((l * NCB + cb) * STAGE_ROWS * 128,
                                STAGE_ROWS * 128)], sem)
            for l in range(LEVEL) for cb in range(NCB)
        ]
        for c in copies:
            c.wait()

        def block_fast(i, carry):
            blk = i * NW + wid

            @pl.when(blk < n_blocks)
            def _():
                base = blk * BLK
                pltpu.sync_copy(x_hbm.at[pl.ds(base, BLK)],
                                xv.at[pl.ds(0, BLK)])
                pltpu.sync_copy(x_hbm.at[pl.ds(n_points + base, BLK)],
                                xv.at[pl.ds(BLK, BLK)])
                for j in range(BLK // LANES):
                    ri = _point_rows(xv[pl.ds(LANES * j, LANES)])
                    ci = _point_cols(xv[pl.ds(BLK + LANES * j, LANES)])
                    cb = lax.shift_right_logical(ci, 7)
                    cr = jnp.bitwise_and(ci, 127)
                    loc = (cb * STAGE_ROWS + (ri - lo)) * 128 + cr
                    for l in range(LEVEL):
                        v = plsc.load_gather(staged, [loc + l * STAGE_SZ])
                        rowsv[pl.ds(l * BLK + LANES * j, LANES)] = v
                for l in range(LEVEL):
                    pltpu.sync_copy(
                        rowsv.at[pl.ds(l * BLK, BLK)],
                        out_hbm.at[pl.ds(l * n_points + base, BLK)])

            return carry

        lax.fori_loop(0, blocks_per_tile, block_fast, 0)

    @pl.when(jnp.logical_not(fast))
    def _():
        def pad(t, carry):
            for l in range(LEVEL):
                idxv4[pl.ds(BLK_PAD * l + BLK + LANES * t, LANES)] = zeros
            return carry

        lax.fori_loop(0, (BLK_PAD - BLK) // LANES, pad, 0)

        def block_slow(i, carry):
            blk = i * NW + wid

            @pl.when(blk < n_blocks)
            def _():
                base = blk * BLK
                pltpu.sync_copy(x_hbm.at[pl.ds(base, BLK)],
                                xv.at[pl.ds(0, BLK)])
                pltpu.sync_copy(x_hbm.at[pl.ds(n_points + base, BLK)],
                                xv.at[pl.ds(BLK, BLK)])

                def jbody(j, c):
                    ri = _point_rows(xv[pl.ds(LANES * j, LANES)])
                    ci = _point_cols(xv[pl.ds(BLK + LANES * j, LANES)])
                    cb = lax.shift_right_logical(ci, 7)
                    cr = jnp.bitwise_and(ci, 127)
                    fidx = (cb * LAT_PAD + ri) * 128 + cr
                    for l in range(LEVEL):
                        idxv4[pl.ds(BLK_PAD * l + LANES * j, LANES)] = (
                            fidx + l * PLANE)
                    return c

                lax.fori_loop(0, BLK // LANES, jbody, 0)
                for l in range(LEVEL):
                    copies = [
                        pltpu.async_copy(
                            tab_hbm.at[idxv4.at[pl.ds(BLK_PAD * l + 128 * k,
                                                      128)]],
                            rows4.at[pl.ds(BLK_PAD * l + 128 * k, 128)], sem)
                        for k in range(BLK_PAD // 128)
                    ]
                    for c in copies:
                        c.wait()
                for l in range(LEVEL):
                    pltpu.sync_copy(
                        rows4.at[pl.ds(l * BLK_PAD, BLK)],
                        out_hbm.at[pl.ds(l * n_points + base, BLK)])

            return carry

        lax.fori_loop(0, blocks_per_tile, block_slow, 0)


def _sc_gather(x, tab1d, n_points):
    return pl.kernel(
        _sc_gather_body,
        out_type=jax.ShapeDtypeStruct((n_points * LEVEL,), jnp.float32),
        mesh=plsc.VectorSubcoreMesh(core_axis_name="c", subcore_axis_name="s"),
        compiler_params=pltpu.CompilerParams(needs_layout_passes=False),
        scratch_types=[
            pltpu.VMEM((2 * BLK,), jnp.float32),
            pltpu.VMEM((LEVEL * STAGE_SZ,), jnp.float32),
            pltpu.VMEM((BLK_PAD * LEVEL,), jnp.int32),
            pltpu.VMEM((BLK_PAD * LEVEL,), jnp.float32),
            pltpu.VMEM((BLK_PAD * LEVEL,), jnp.float32),
            pltpu.VMEM((LANES,), jnp.int32),
            pltpu.VMEM((LANES,), jnp.int32),
            pltpu.SemaphoreType.DMA,
        ],
    )(x, tab1d)


def kernel(x, grid0, grid1, grid2, grid3, north_pole_param, south_pole_param):
    n_points = x.shape[0]
    planar = _build_table(grid0[0, 0], grid1[0, 0], grid2[0, 0], grid3[0, 0],
                          north_pole_param, south_pole_param)
    flat = _sc_gather(x.T.reshape(-1), planar.reshape(-1), n_points)
    return flat.reshape(LEVEL, n_points).T


# speedup vs baseline: 10.9085x; 1.0891x over previous
import math

import numpy as np

import jax
import jax.numpy as jnp
from jax import lax
from jax.experimental import pallas as pl
from jax.experimental.pallas import tpu as pltpu
from jax.experimental.pallas import tpu_sc as plsc

LAT = 721
LON = 1440
LEVEL = 4
INV_RES = 4.0
N_ROWS = LAT * LON
LAT_PAD = 728
LON_PAD = 1536
NCB = LON_PAD // 128
CB_SZ = LAT_PAD * 128
PLANE = NCB * CB_SZ

NC = 2
NS = 16
NW = NC * NS
LANES = 16

BLK = 1600
BLK_PAD = 1664
STAGE_ROWS = 16
NCB_STAGE = STAGE_ROWS
STAGE_SZ = NCB * STAGE_ROWS * 128
MAGIC = 12582912.0


def _interp_matrix(src, dst):
    scale = dst / src
    sample = (np.arange(dst, dtype=np.float64) + 0.5) / scale - 0.5
    j = np.arange(src, dtype=np.float64)[None, :]
    w = np.maximum(0.0, 1.0 - np.abs(sample[:, None] - j))
    w = w / np.sum(w, axis=1, keepdims=True)
    return w.astype(np.float32)


def _table_body(g0, g1, g2, g3, r1, c1t, r2, c2t, r3, c3t, npb, spb,
                out_ref):
    rows = lax.broadcasted_iota(jnp.int32, (LAT_PAD, LON_PAD), 0)
    u0 = jnp.concatenate(
        [jnp.concatenate(
            [g0[...], jnp.zeros((LAT, LON_PAD - LON), jnp.float32)], axis=1),
         jnp.zeros((LAT_PAD - LAT, LON_PAD), jnp.float32)], axis=0)
    us = [u0,
          jnp.dot(r1[...],
                  jnp.dot(g1[...], c1t[...],
                          preferred_element_type=jnp.float32),
                  preferred_element_type=jnp.float32),
          jnp.dot(r2[...],
                  jnp.dot(g2[...], c2t[...],
                          preferred_element_type=jnp.float32),
                  preferred_element_type=jnp.float32),
          jnp.dot(r3[...],
                  jnp.dot(g3[...], c3t[...],
                          preferred_element_type=jnp.float32),
                  preferred_element_type=jnp.float32)]
    for l in range(LEVEL):
        u = jnp.where(rows == 0, spb[l:l + 1, :], us[l])
        u = jnp.where(rows == LAT - 1, npb[l:l + 1, :], u)
        for cb in range(NCB):
            out_ref[pl.ds((l * NCB + cb) * LAT_PAD, LAT_PAD), :] = (
                u[:, 128 * cb:128 * (cb + 1)])


def _build_table(g0, g1, g2, g3, north, south):
    shapes = [(int(math.ceil(LAT / 2 ** i)), int(math.ceil(LON / 2 ** i)))
              for i in range(LEVEL)]
    mats = []
    for i in (1, 2, 3):
        h, w = shapes[i]
        r = np.zeros((LAT_PAD, h), np.float32)
        r[:LAT] = _interp_matrix(h, LAT)
        mats.append(r)
        ct = np.zeros((w, LON_PAD), np.float32)
        ct[:, :LON] = _interp_matrix(w, LON).T
        mats.append(ct)
    npb = jnp.broadcast_to(north[:, None], (LEVEL, LON_PAD))
    spb = jnp.broadcast_to(south[:, None], (LEVEL, LON_PAD))
    return pl.pallas_call(
        _table_body,
        out_shape=jax.ShapeDtypeStruct((LEVEL * NCB * LAT_PAD, 128),
                                       jnp.float32),
    )(g0, g1, g2, g3, *mats, npb, spb)


def _point_rows(latv):
    v = (90.0 - latv) * INV_RES
    r = (v + MAGIC) - MAGIC
    r = jnp.minimum(jnp.maximum(r, 0.0), float(LAT - 1))
    return r.astype(jnp.int32)


def _point_cols(lonv):
    v = lonv * INV_RES
    r = (v + MAGIC) - MAGIC
    r = jnp.minimum(jnp.maximum(r, 0.0), float(LON - 1))
    return r.astype(jnp.int32)


def _sc_gather_body(x_hbm, tab_hbm, out_hbm,
                    xv, staged, idxv4, rows4, rowsv, sem):
    wid = lax.axis_index("s") * NC + lax.axis_index("c")
    n_points = x_hbm.shape[0] // 2
    n_blocks = n_points // BLK
    blocks_per_tile = (n_blocks + NW - 1) // NW
    iota = lax.iota(jnp.int32, LANES)
    zeros = jnp.zeros((LANES,), jnp.int32)

    for t in range((BLK_PAD - BLK) // LANES):
        for l in range(LEVEL):
            idxv4[pl.ds(BLK_PAD * l + BLK + LANES * t, LANES)] = zeros

    pltpu.sync_copy(x_hbm.at[pl.ds(wid * BLK, BLK)], xv.at[pl.ds(0, BLK)])
    rmin = jnp.full((LANES,), LAT - 1, jnp.int32)
    rmax = jnp.zeros((LANES,), jnp.int32)
    for j in range(BLK // LANES):
        ri = _point_rows(xv[pl.ds(LANES * j, LANES)])
        rmin = jnp.minimum(rmin, ri)
        rmax = jnp.maximum(rmax, ri)
    lo = jnp.minimum(jnp.maximum(jnp.min(rmin) - 5, 0), LAT - STAGE_ROWS)
    hi_cap = lo + STAGE_ROWS

    copies = [
        pltpu.async_copy(
            tab_hbm.at[pl.ds((l * NCB + cb) * CB_SZ + lo * 128,
                             STAGE_ROWS * 128)],
            staged.at[pl.ds((l * NCB + cb) * STAGE_ROWS * 128,
                            STAGE_ROWS * 128)], sem)
        for l in range(LEVEL) for cb in range(NCB)
    ]
    for c in copies:
        c.wait()

    def block(i, carry):
        blk = i * NW + wid

        @pl.when(blk < n_blocks)
        def _():
            base = blk * BLK
            pltpu.sync_copy(x_hbm.at[pl.ds(base, BLK)], xv.at[pl.ds(0, BLK)])
            pltpu.sync_copy(x_hbm.at[pl.ds(n_points + base, BLK)],
                            xv.at[pl.ds(BLK, BLK)])
            bmin = jnp.full((LANES,), LAT - 1, jnp.int32)
            bmax = jnp.zeros((LANES,), jnp.int32)
            for j in range(BLK // LANES):
                ri = _point_rows(xv[pl.ds(LANES * j, LANES)])
                ci = _point_cols(xv[pl.ds(BLK + LANES * j, LANES)])
                bmin = jnp.minimum(bmin, ri)
                bmax = jnp.maximum(bmax, ri)
                cb = lax.shift_right_logical(ci, 7)
                cr = jnp.bitwise_and(ci, 127)
                rloc = jnp.minimum(jnp.maximum(ri - lo, 0), STAGE_ROWS - 1)
                loc = (cb * STAGE_ROWS + rloc) * 128 + cr
                for l in range(LEVEL):
                    v = plsc.load_gather(staged, [loc + l * STAGE_SZ])
                    rowsv[pl.ds(l * BLK + LANES * j, LANES)] = v
            ok = jnp.logical_and(jnp.min(bmin) >= lo, jnp.max(bmax) < hi_cap)

            @pl.when(ok)
            def _():
                for l in range(LEVEL):
                    pltpu.sync_copy(
                        rowsv.at[pl.ds(l * BLK, BLK)],
                        out_hbm.at[pl.ds(l * n_points + base, BLK)])

            @pl.when(jnp.logical_not(ok))
            def _():
                def jbody(j, c):
                    ri = _point_rows(xv[pl.ds(LANES * j, LANES)])
                    ci = _point_cols(xv[pl.ds(BLK + LANES * j, LANES)])
                    cb = lax.shift_right_logical(ci, 7)
                    cr = jnp.bitwise_and(ci, 127)
                    fidx = (cb * LAT_PAD + ri) * 128 + cr
                    for l in range(LEVEL):
                        idxv4[pl.ds(BLK_PAD * l + LANES * j, LANES)] = (
                            fidx + l * PLANE)
                    return c

                lax.fori_loop(0, BLK // LANES, jbody, 0)
                for l in range(LEVEL):
                    copies = [
                        pltpu.async_copy(
                            tab_hbm.at[idxv4.at[pl.ds(BLK_PAD * l + 128 * k,
                                                      128)]],
                            rows4.at[pl.ds(BLK_PAD * l + 128 * k, 128)], sem)
                        for k in range(BLK_PAD // 128)
                    ]
                    for c in copies:
                        c.wait()
                for l in range(LEVEL):
                    pltpu.sync_copy(
                        rows4.at[pl.ds(l * BLK_PAD, BLK)],
                        out_hbm.at[pl.ds(l * n_points + base, BLK)])

        return carry

    lax.fori_loop(0, blocks_per_tile, block, 0)


def _sc_gather(x, tab1d, n_points):
    return pl.kernel(
        _sc_gather_body,
        out_type=jax.ShapeDtypeStruct((n_points * LEVEL,), jnp.float32),
        mesh=plsc.VectorSubcoreMesh(core_axis_name="c", subcore_axis_name="s"),
        compiler_params=pltpu.CompilerParams(needs_layout_passes=False),
        scratch_types=[
            pltpu.VMEM((2 * BLK,), jnp.float32),
            pltpu.VMEM((LEVEL * STAGE_SZ,), jnp.float32),
            pltpu.VMEM((BLK_PAD * LEVEL,), jnp.int32),
            pltpu.VMEM((BLK_PAD * LEVEL,), jnp.float32),
            pltpu.VMEM((LEVEL * BLK,), jnp.float32),
            pltpu.SemaphoreType.DMA,
        ],
    )(x, tab1d)


def kernel(x, grid0, grid1, grid2, grid3, north_pole_param, south_pole_param):
    n_points = x.shape[0]
    planar = _build_table(grid0[0, 0], grid1[0, 0], grid2[0, 0], grid3[0, 0],
                          north_pole_param, south_pole_param)
    flat = _sc_gather(x.T.reshape(-1), planar.reshape(-1), n_points)
    return flat.reshape(LEVEL, n_points).T


# speedup vs baseline: 14.2203x; 1.3036x over previous
import math

import numpy as np

import jax
import jax.numpy as jnp
from jax import lax
from jax.experimental import pallas as pl
from jax.experimental.pallas import tpu as pltpu
from jax.experimental.pallas import tpu_sc as plsc

LAT = 721
LON = 1440
LEVEL = 4
INV_RES = 4.0
N_ROWS = LAT * LON
LAT_PAD = 728
LON_PAD = 1536
NCB = LON_PAD // 128
CB_SZ = LAT_PAD * 128
PLANE = NCB * CB_SZ

NC = 2
NS = 16
NW = NC * NS
LANES = 16

BLK = 1600
BLK_PAD = 1664
STAGE_ROWS = 16
NCB_STAGE = STAGE_ROWS
STAGE_SZ = NCB * STAGE_ROWS * 128
MAGIC = 12582912.0


def _interp_matrix(src, dst):
    scale = dst / src
    sample = (np.arange(dst, dtype=np.float64) + 0.5) / scale - 0.5
    j = np.arange(src, dtype=np.float64)[None, :]
    w = np.maximum(0.0, 1.0 - np.abs(sample[:, None] - j))
    w = w / np.sum(w, axis=1, keepdims=True)
    return w.astype(np.float32)


def _table_body(g0, g1, g2, g3, r1, c1t, r2, c2t, r3, c3t, npb, spb,
                out_ref):
    rows = lax.broadcasted_iota(jnp.int32, (LAT_PAD, LON_PAD), 0)
    u0 = jnp.concatenate(
        [jnp.concatenate(
            [g0[...], jnp.zeros((LAT, LON_PAD - LON), jnp.float32)], axis=1),
         jnp.zeros((LAT_PAD - LAT, LON_PAD), jnp.float32)], axis=0)
    us = [u0,
          jnp.dot(r1[...],
                  jnp.dot(g1[...], c1t[...],
                          preferred_element_type=jnp.float32),
                  preferred_element_type=jnp.float32),
          jnp.dot(r2[...],
                  jnp.dot(g2[...], c2t[...],
                          preferred_element_type=jnp.float32),
                  preferred_element_type=jnp.float32),
          jnp.dot(r3[...],
                  jnp.dot(g3[...], c3t[...],
                          preferred_element_type=jnp.float32),
                  preferred_element_type=jnp.float32)]
    for l in range(LEVEL):
        u = jnp.where(rows == 0, spb[l:l + 1, :], us[l])
        u = jnp.where(rows == LAT - 1, npb[l:l + 1, :], u)
        for cb in range(NCB):
            out_ref[pl.ds((l * NCB + cb) * LAT_PAD, LAT_PAD), :] = (
                u[:, 128 * cb:128 * (cb + 1)])


def _build_table(g0, g1, g2, g3, north, south):
    shapes = [(int(math.ceil(LAT / 2 ** i)), int(math.ceil(LON / 2 ** i)))
              for i in range(LEVEL)]
    mats = []
    for i in (1, 2, 3):
        h, w = shapes[i]
        r = np.zeros((LAT_PAD, h), np.float32)
        r[:LAT] = _interp_matrix(h, LAT)
        mats.append(r)
        ct = np.zeros((w, LON_PAD), np.float32)
        ct[:, :LON] = _interp_matrix(w, LON).T
        mats.append(ct)
    npb = jnp.broadcast_to(north[:, None], (LEVEL, LON_PAD))
    spb = jnp.broadcast_to(south[:, None], (LEVEL, LON_PAD))
    return pl.pallas_call(
        _table_body,
        out_shape=jax.ShapeDtypeStruct((LEVEL * NCB * LAT_PAD, 128),
                                       jnp.float32),
    )(g0, g1, g2, g3, *mats, npb, spb)


def _point_rows(latv):
    v = (90.0 - latv) * INV_RES
    r = (v + MAGIC) - MAGIC
    r = jnp.minimum(jnp.maximum(r, 0.0), float(LAT - 1))
    return r.astype(jnp.int32)


def _point_cols(lonv):
    v = lonv * INV_RES
    r = (v + MAGIC) - MAGIC
    r = jnp.minimum(jnp.maximum(r, 0.0), float(LON - 1))
    return r.astype(jnp.int32)


def _sc_gather_body(x_hbm, tab_hbm, out_hbm,
                    xv, staged, idxv4, rows4, rowsv, sem):
    wid = lax.axis_index("s") * NC + lax.axis_index("c")
    n_points = x_hbm.shape[0] // 2
    n_blocks = n_points // BLK
    blocks_per_tile = (n_blocks + NW - 1) // NW
    iota = lax.iota(jnp.int32, LANES)
    zeros = jnp.zeros((LANES,), jnp.int32)

    for t in range((BLK_PAD - BLK) // LANES):
        for l in range(LEVEL):
            idxv4[pl.ds(BLK_PAD * l + BLK + LANES * t, LANES)] = zeros

    pltpu.sync_copy(x_hbm.at[pl.ds(wid * BLK, BLK)], xv.at[pl.ds(0, BLK)])

    @plsc.parallel_loop(0, BLK, LANES, unroll=4,
                        carry=jnp.full((LANES,), LAT - 1, jnp.int32))
    def rmin(p, acc):
        return jnp.minimum(acc, _point_rows(xv[pl.ds(p, LANES)]))

    lo = jnp.minimum(jnp.maximum(jnp.min(rmin) - 5, 0), LAT - STAGE_ROWS)
    hi_cap = lo + STAGE_ROWS

    copies = [
        pltpu.async_copy(
            tab_hbm.at[pl.ds((l * NCB + cb) * CB_SZ + lo * 128,
                             STAGE_ROWS * 128)],
            staged.at[pl.ds((l * NCB + cb) * STAGE_ROWS * 128,
                            STAGE_ROWS * 128)], sem)
        for l in range(LEVEL) for cb in range(NCB)
    ]
    for c in copies:
        c.wait()

    def block(i, carry):
        blk = i * NW + wid

        @pl.when(blk < n_blocks)
        def _():
            base = blk * BLK
            pltpu.sync_copy(x_hbm.at[pl.ds(base, BLK)], xv.at[pl.ds(0, BLK)])
            pltpu.sync_copy(x_hbm.at[pl.ds(n_points + base, BLK)],
                            xv.at[pl.ds(BLK, BLK)])
            carry0 = (jnp.full((LANES,), LAT - 1, jnp.int32),
                      jnp.zeros((LANES,), jnp.int32))

            @plsc.parallel_loop(0, BLK, LANES, unroll=4, carry=carry0)
            def mm(p, acc):
                bmin, bmax = acc
                ri = _point_rows(xv[pl.ds(p, LANES)])
                ci = _point_cols(xv[pl.ds(BLK + p, LANES)])
                cb = lax.shift_right_logical(ci, 7)
                cr = jnp.bitwise_and(ci, 127)
                rloc = jnp.minimum(jnp.maximum(ri - lo, 0), STAGE_ROWS - 1)
                loc = (cb * STAGE_ROWS + rloc) * 128 + cr
                for l in range(LEVEL):
                    v = plsc.load_gather(staged, [loc + l * STAGE_SZ])
                    rowsv[pl.ds(l * BLK + p, LANES)] = v
                return (jnp.minimum(bmin, ri), jnp.maximum(bmax, ri))

            bmin, bmax = mm
            ok = jnp.logical_and(jnp.min(bmin) >= lo, jnp.max(bmax) < hi_cap)

            @pl.when(ok)
            def _():
                for l in range(LEVEL):
                    pltpu.sync_copy(
                        rowsv.at[pl.ds(l * BLK, BLK)],
                        out_hbm.at[pl.ds(l * n_points + base, BLK)])

            @pl.when(jnp.logical_not(ok))
            def _():
                def jbody(j, c):
                    ri = _point_rows(xv[pl.ds(LANES * j, LANES)])
                    ci = _point_cols(xv[pl.ds(BLK + LANES * j, LANES)])
                    cb = lax.shift_right_logical(ci, 7)
                    cr = jnp.bitwise_and(ci, 127)
                    fidx = (cb * LAT_PAD + ri) * 128 + cr
                    for l in range(LEVEL):
                        idxv4[pl.ds(BLK_PAD * l + LANES * j, LANES)] = (
                            fidx + l * PLANE)
                    return c

                lax.fori_loop(0, BLK // LANES, jbody, 0)
                for l in range(LEVEL):
                    copies = [
                        pltpu.async_copy(
                            tab_hbm.at[idxv4.at[pl.ds(BLK_PAD * l + 128 * k,
                                                      128)]],
                            rows4.at[pl.ds(BLK_PAD * l + 128 * k, 128)], sem)
                        for k in range(BLK_PAD // 128)
                    ]
                    for c in copies:
                        c.wait()
                for l in range(LEVEL):
                    pltpu.sync_copy(
                        rows4.at[pl.ds(l * BLK_PAD, BLK)],
                        out_hbm.at[pl.ds(l * n_points + base, BLK)])

        return carry

    lax.fori_loop(0, blocks_per_tile, block, 0)


def _sc_gather(x, tab1d, n_points):
    return pl.kernel(
        _sc_gather_body,
        out_type=jax.ShapeDtypeStruct((n_points * LEVEL,), jnp.float32),
        mesh=plsc.VectorSubcoreMesh(core_axis_name="c", subcore_axis_name="s"),
        compiler_params=pltpu.CompilerParams(needs_layout_passes=False),
        scratch_types=[
            pltpu.VMEM((2 * BLK,), jnp.float32),
            pltpu.VMEM((LEVEL * STAGE_SZ,), jnp.float32),
            pltpu.VMEM((BLK_PAD * LEVEL,), jnp.int32),
            pltpu.VMEM((BLK_PAD * LEVEL,), jnp.float32),
            pltpu.VMEM((LEVEL * BLK,), jnp.float32),
            pltpu.SemaphoreType.DMA,
        ],
    )(x, tab1d)


def kernel(x, grid0, grid1, grid2, grid3, north_pole_param, south_pole_param):
    n_points = x.shape[0]
    planar = _build_table(grid0[0, 0], grid1[0, 0], grid2[0, 0], grid3[0, 0],
                          north_pole_param, south_pole_param)
    flat = _sc_gather(x.T.reshape(-1), planar.reshape(-1), n_points)
    return flat.reshape(LEVEL, n_points).T
